# baseline (speedup 1.0000x reference)
import functools

import jax
import jax.numpy as jnp
from jax import lax
from jax.experimental import pallas as pl
from jax.experimental.pallas import tpu as pltpu
from jax.experimental.pallas import tpu_sc as plsc

B = 16384
D = 64
OUT_D = 2 * D
VOCAB = 1000000
L = 16
CT = 12
WLANES = CT * 128
CAP = 1056
NBUF = 64


def kernel(input, input_position, input_table, pos_table):
    idx = input.astype(jnp.int32)
    table_t = jnp.transpose(input_table)
    posrow = jnp.concatenate(
        [jnp.take(pos_table, jnp.asarray(input_position), axis=0),
         jnp.zeros((D,), jnp.float32)])

    info = plsc.get_sparse_core_info()
    nw = info.num_cores * info.num_subcores
    per_w = VOCAB // nw
    n_win = -(-(per_w + 127) // WLANES)
    mesh = plsc.VectorSubcoreMesh(core_axis_name="c", subcore_axis_name="s")

    @functools.partial(
        pl.kernel,
        out_type=jax.ShapeDtypeStruct((B, OUT_D), jnp.float32),
        mesh=mesh,
        compiler_params=pltpu.CompilerParams(
            needs_layout_passes=False, disable_bounds_checks=True),
        scratch_types=[
            pltpu.VMEM((B,), jnp.int32),
            pltpu.VMEM((CAP,), jnp.int32),
            pltpu.VMEM((CAP,), jnp.int32),
            pltpu.VMEM((D, WLANES), jnp.float32),
            pltpu.VMEM((NBUF, OUT_D), jnp.float32),
            pltpu.VMEM((OUT_D,), jnp.float32),
            pltpu.SemaphoreType.DMA,
            pltpu.SemaphoreType.DMA,
        ],
    )
    def sc_kernel(idx_hbm, pos_hbm, table_hbm, out_hbm,
                  idx_v, mv_v, mi_v, stage_v, rows_v, prow_v, sem_s, sem_o):
        wid = lax.axis_index("s") * info.num_cores + lax.axis_index("c")
        lo = wid * per_w
        hi = lo + per_w

        pltpu.sync_copy(idx_hbm, idx_v)
        pltpu.sync_copy(pos_hbm, prow_v)
        pvs = [prow_v[pl.ds(L * j, L)] for j in range(D // L)]
        lanes = lax.iota(jnp.int32, L)

        def scan(k, ptr):
            iv = idx_v[pl.ds(k * L, L)]
            m = jnp.logical_and(iv >= lo, iv < hi)
            nhit = plsc.all_reduce_population_count(m)[0]

            @pl.when(nhit > 0)
            def _():
                key = jnp.where(m, 32, 0) + (15 - lanes)
                mv_v[pl.ds(ptr, L)] = plsc.sort_key_val(
                    key, iv, descending=True)[1]
                mi_v[pl.ds(ptr, L)] = plsc.sort_key_val(
                    key, k * L + lanes, descending=True)[1]
            return ptr + nhit
        nmatch = lax.fori_loop(0, B // L, scan, jnp.int32(0))

        t0l = lo - lax.rem(lo, 128)

        last_full = ((VOCAB // 128) * 128) - WLANES

        def emit_window(o_k, width, cnt):
            def emit(q, cnt_q):
                gl = q * L + lanes
                mvv = mv_v[pl.ds(q * L, L)]
                hit = jnp.logical_and(
                    gl < nmatch,
                    jnp.logical_and(mvv >= o_k, mvv < o_k + width))
                nhit = plsc.all_reduce_population_count(hit)[0]
                hit32 = hit.astype(jnp.int32)

                @pl.when(nhit > 0)
                def _():
                    miv = mi_v[pl.ds(q * L, L)]
                    for r in range(L):
                        @pl.when(hit32[r] > 0)
                        def _():
                            col = jnp.full((L,), mvv[r] - o_k, jnp.int32)
                            pre_r = plsc.all_reduce_population_count(
                                jnp.logical_and(hit, lanes <= r))[0]
                            slot = lax.rem(cnt_q + pre_r - 1, NBUF)
                            for j in range(D // L):
                                g = plsc.load_gather(
                                    stage_v, [L * j + lanes, col])
                                rows_v[slot, pl.ds(L * j, L)] = g
                                rows_v[slot, pl.ds(D + L * j, L)] = pvs[j]
                            pltpu.async_copy(
                                rows_v.at[slot], out_hbm.at[miv[r]], sem_o)
                return cnt_q + nhit

            cnt2 = lax.fori_loop(0, CAP // L, emit, cnt)

            def drain(_, c):
                pltpu.make_async_copy(
                    rows_v.at[0], out_hbm.at[0], sem_o).wait()
                return c
            lax.fori_loop(0, cnt2 - cnt, drain, jnp.int32(0))
            return cnt2

        def window(k, cnt):
            o_k = jnp.minimum(t0l + k * WLANES, jnp.int32(last_full))
            o_k = pl.multiple_of(o_k, 128)
            for cb in range(D // 8):
                pltpu.async_copy(
                    table_hbm.at[pl.ds(cb * 8, 8), pl.ds(o_k, WLANES)],
                    stage_v.at[pl.ds(cb * 8, 8), :], sem_s)
            for cb in range(D // 8):
                pltpu.make_async_copy(
                    table_hbm.at[pl.ds(0, 8), pl.ds(0, WLANES)],
                    stage_v.at[pl.ds(0, 8), :], sem_s).wait()
            return emit_window(o_k, WLANES, cnt)

        cnt_main = lax.fori_loop(0, n_win, window, jnp.int32(0))

        tail = (VOCAB // 128) * 128
        @pl.when(wid == nw - 1)
        def _():
            tailo = pl.multiple_of(hi * 0 + tail, 128)
            for cb in range(D // 8):
                pltpu.async_copy(
                    table_hbm.at[pl.ds(cb * 8, 8), pl.ds(tailo, 128)],
                    stage_v.at[pl.ds(cb * 8, 8), pl.ds(0, 128)],
                    sem_s)
            for cb in range(D // 8):
                pltpu.make_async_copy(
                    table_hbm.at[pl.ds(0, 8), pl.ds(tailo, 128)],
                    stage_v.at[pl.ds(0, 8), pl.ds(0, 128)],
                    sem_s).wait()
            emit_window(jnp.int32(tail), 128, cnt_main)

    return sc_kernel(idx, posrow, table_t)


# speedup vs baseline: 1.3373x; 1.3373x over previous
import functools

import jax
import jax.numpy as jnp
from jax import lax
from jax.experimental import pallas as pl
from jax.experimental.pallas import tpu as pltpu
from jax.experimental.pallas import tpu_sc as plsc

B = 16384
D = 64
OUT_D = 2 * D
VOCAB = 1000000
L = 16
WLANES = 768
CAP = 1056
NBUF = 64


def kernel(input, input_position, input_table, pos_table):
    idx = input.astype(jnp.int32)
    table_t = jnp.transpose(input_table)
    posrow = jnp.concatenate(
        [jnp.take(pos_table, jnp.asarray(input_position), axis=0),
         jnp.zeros((D,), jnp.float32)])

    info = plsc.get_sparse_core_info()
    nw = info.num_cores * info.num_subcores
    per_w = VOCAB // nw
    n_win = -(-(per_w + 127) // WLANES)
    mesh = plsc.VectorSubcoreMesh(core_axis_name="c", subcore_axis_name="s")

    def lane_bcast(x, r):
        return lax.gather(
            x, jnp.full((L, 1), r, jnp.int32),
            lax.GatherDimensionNumbers(
                offset_dims=(), collapsed_slice_dims=(0,),
                start_index_map=(0,)),
            (1,), mode=lax.GatherScatterMode.PROMISE_IN_BOUNDS)

    @functools.partial(
        pl.kernel,
        out_type=jax.ShapeDtypeStruct((B, OUT_D), jnp.float32),
        mesh=mesh,
        compiler_params=pltpu.CompilerParams(
            needs_layout_passes=False, disable_bounds_checks=True),
        scratch_types=[
            pltpu.VMEM((B,), jnp.int32),
            pltpu.VMEM((CAP,), jnp.int32),
            pltpu.VMEM((CAP,), jnp.int32),
            pltpu.VMEM((D, 2 * WLANES), jnp.float32),
            pltpu.VMEM((NBUF, OUT_D), jnp.float32),
            pltpu.VMEM((OUT_D,), jnp.float32),
            pltpu.SemaphoreType.DMA,
            pltpu.SemaphoreType.DMA,
        ],
    )
    def sc_kernel(idx_hbm, pos_hbm, table_hbm, out_hbm,
                  idx_v, mv_v, mi_v, stage_v, rows_v, prow_v, sem_s, sem_o):
        wid = lax.axis_index("s") * info.num_cores + lax.axis_index("c")
        lo = wid * per_w
        hi = lo + per_w

        pltpu.sync_copy(idx_hbm, idx_v)
        pltpu.sync_copy(pos_hbm, prow_v)
        pvs = [prow_v[pl.ds(L * j, L)] for j in range(D // L)]
        lanes = lax.iota(jnp.int32, L)

        def prefill(s, carry):
            for j in range(D // L):
                rows_v[s, pl.ds(D + L * j, L)] = pvs[j]
            return carry
        lax.fori_loop(0, NBUF, prefill, 0)

        def scan(k, ptr):
            iv = idx_v[pl.ds(k * L, L)]
            m = jnp.logical_and(iv >= lo, iv < hi)

            def on_hit():
                key = jnp.where(m, 32, 0) + (15 - lanes)
                mv_v[pl.ds(ptr, L)] = plsc.sort_key_val(
                    key, iv, descending=True)[1]
                mi_v[pl.ds(ptr, L)] = plsc.sort_key_val(
                    key, k * L + lanes, descending=True)[1]
                return ptr + plsc.all_reduce_population_count(m)[0]
            return lax.cond(jnp.any(m), on_hit, lambda: ptr)
        nmatch = lax.fori_loop(0, B // L, scan, jnp.int32(0))

        t0l = lo - lax.rem(lo, 128)
        last_full = ((VOCAB // 128) * 128) - WLANES

        def win_off(k):
            o = jnp.minimum(t0l + k * WLANES, jnp.int32(last_full))
            return pl.multiple_of(o, 128)

        def start(k):
            o_k = win_off(k)
            half = lax.rem(k, 2) * WLANES
            for cb in range(D // 8):
                pltpu.async_copy(
                    table_hbm.at[pl.ds(cb * 8, 8), pl.ds(o_k, WLANES)],
                    stage_v.at[pl.ds(cb * 8, 8), pl.ds(half, WLANES)], sem_s)

        def wait_stage():
            for cb in range(D // 8):
                pltpu.make_async_copy(
                    table_hbm.at[pl.ds(0, 8), pl.ds(0, WLANES)],
                    stage_v.at[pl.ds(0, 8), pl.ds(0, WLANES)], sem_s).wait()

        def emit_window(o_k, base, width, cnt):
            def emit(q, cnt_q):
                gl = q * L + lanes
                mvv = mv_v[pl.ds(q * L, L)]
                hit = jnp.logical_and(
                    gl < nmatch,
                    jnp.logical_and(mvv >= o_k, mvv < o_k + width))

                def on_hit():
                    key = jnp.where(hit, 32, 0) + (15 - lanes)
                    mv_s = plsc.sort_key_val(key, mvv, descending=True)[1]
                    mi_s = plsc.sort_key_val(
                        key, mi_v[pl.ds(q * L, L)], descending=True)[1]
                    nh = plsc.all_reduce_population_count(hit)[0]
                    colbase = mv_s - o_k + base
                    for r in range(L):
                        @pl.when(r < nh)
                        def _():
                            col = lane_bcast(colbase, r)
                            slot = lax.rem(cnt_q + r, NBUF)
                            for j in range(D // L):
                                rows_v[slot, pl.ds(L * j, L)] = (
                                    plsc.load_gather(
                                        stage_v, [L * j + lanes, col]))
                            pltpu.async_copy(
                                rows_v.at[slot], out_hbm.at[mi_s[r]], sem_o)
                    return cnt_q + nh
                return lax.cond(jnp.any(hit), on_hit, lambda: cnt_q)

            cnt2 = lax.fori_loop(0, CAP // L, emit, cnt)

            def drain(_, c):
                pltpu.make_async_copy(
                    rows_v.at[0], out_hbm.at[0], sem_o).wait()
                return c
            lax.fori_loop(0, cnt2 - cnt, drain, jnp.int32(0))
            return cnt2

        start(0)

        def window(k, cnt):
            wait_stage()

            @pl.when(k + 1 < n_win)
            def _():
                start(k + 1)
            return emit_window(win_off(k), lax.rem(k, 2) * WLANES,
                               WLANES, cnt)

        cnt_main = lax.fori_loop(0, n_win, window, jnp.int32(0))

        tail = (VOCAB // 128) * 128
        @pl.when(wid == nw - 1)
        def _():
            tailo = pl.multiple_of(hi * 0 + tail, 128)
            for cb in range(D // 8):
                pltpu.async_copy(
                    table_hbm.at[pl.ds(cb * 8, 8), pl.ds(tailo, 128)],
                    stage_v.at[pl.ds(cb * 8, 8), pl.ds(0, 128)], sem_s)
            for cb in range(D // 8):
                pltpu.make_async_copy(
                    table_hbm.at[pl.ds(0, 8), pl.ds(tailo, 128)],
                    stage_v.at[pl.ds(0, 8), pl.ds(0, 128)], sem_s).wait()
            emit_window(jnp.int32(tail), 0, 128, cnt_main)

    return sc_kernel(idx, posrow, table_t)


# speedup vs baseline: 1.4234x; 1.0644x over previous
import functools

import jax
import jax.numpy as jnp
from jax import lax
from jax.experimental import pallas as pl
from jax.experimental.pallas import tpu as pltpu
from jax.experimental.pallas import tpu_sc as plsc

B = 16384
D = 64
OUT_D = 2 * D
VOCAB = 1000000
L = 16
WLANES = 640
CAP = 768
NBUF = 128


def kernel(input, input_position, input_table, pos_table):
    idx = input.astype(jnp.int32)
    table_t = jnp.transpose(input_table)
    posrow = jnp.concatenate(
        [jnp.take(pos_table, jnp.asarray(input_position), axis=0),
         jnp.zeros((D,), jnp.float32)])

    info = plsc.get_sparse_core_info()
    nw = info.num_cores * info.num_subcores
    per_w = VOCAB // nw
    n_win = -(-(per_w + 127) // WLANES)
    mesh = plsc.VectorSubcoreMesh(core_axis_name="c", subcore_axis_name="s")

    def lane_bcast(x, r):
        return lax.gather(
            x, jnp.full((L, 1), r, jnp.int32),
            lax.GatherDimensionNumbers(
                offset_dims=(), collapsed_slice_dims=(0,),
                start_index_map=(0,)),
            (1,), mode=lax.GatherScatterMode.PROMISE_IN_BOUNDS)

    @functools.partial(
        pl.kernel,
        out_type=jax.ShapeDtypeStruct((B, OUT_D), jnp.float32),
        mesh=mesh,
        compiler_params=pltpu.CompilerParams(
            needs_layout_passes=False, disable_bounds_checks=True),
        scratch_types=[
            pltpu.VMEM((B,), jnp.int32),
            pltpu.VMEM((CAP,), jnp.int32),
            pltpu.VMEM((CAP,), jnp.int32),
            pltpu.VMEM((D, 2 * WLANES), jnp.float32),
            pltpu.VMEM((NBUF, OUT_D), jnp.float32),
            pltpu.VMEM((OUT_D,), jnp.float32),
            pltpu.SemaphoreType.DMA,
            pltpu.SemaphoreType.DMA,
        ],
    )
    def sc_kernel(idx_hbm, pos_hbm, table_hbm, out_hbm,
                  idx_v, mv_v, mi_v, stage_v, rows_v, prow_v, sem_s, sem_o):
        wid = lax.axis_index("s") * info.num_cores + lax.axis_index("c")
        lo = wid * per_w
        hi = lo + per_w

        pltpu.sync_copy(idx_hbm, idx_v)
        pltpu.sync_copy(pos_hbm, prow_v)
        pvs = [prow_v[pl.ds(L * j, L)] for j in range(D // L)]
        lanes = lax.iota(jnp.int32, L)

        def prefill(s, carry):
            for j in range(D // L):
                rows_v[s, pl.ds(D + L * j, L)] = pvs[j]
            return carry
        lax.fori_loop(0, NBUF, prefill, 0)

        def scan(k, ptr):
            iv = idx_v[pl.ds(k * L, L)]
            m = jnp.logical_and(iv >= lo, iv < hi)

            def on_hit():
                key = jnp.where(m, 32, 0) + (15 - lanes)
                mv_v[pl.ds(ptr, L)] = plsc.sort_key_val(
                    key, iv, descending=True)[1]
                mi_v[pl.ds(ptr, L)] = plsc.sort_key_val(
                    key, k * L + lanes, descending=True)[1]
                return ptr + plsc.all_reduce_population_count(m)[0]
            return lax.cond(jnp.any(m), on_hit, lambda: ptr)
        t0l = lo - lax.rem(lo, 128)
        last_full = ((VOCAB // 128) * 128) - WLANES

        def win_off(k):
            o = jnp.minimum(t0l + k * WLANES, jnp.int32(last_full))
            return pl.multiple_of(o, 128)

        def start(k):
            o_k = win_off(k)
            half = lax.rem(k, 2) * WLANES
            for cb in range(D // 8):
                pltpu.async_copy(
                    table_hbm.at[pl.ds(cb * 8, 8), pl.ds(o_k, WLANES)],
                    stage_v.at[pl.ds(cb * 8, 8), pl.ds(half, WLANES)], sem_s)

        def wait_stage():
            for cb in range(D // 8):
                pltpu.make_async_copy(
                    table_hbm.at[pl.ds(0, 8), pl.ds(0, WLANES)],
                    stage_v.at[pl.ds(0, 8), pl.ds(0, WLANES)], sem_s).wait()

        start(0)
        start(1)
        nmatch = lax.fori_loop(0, B // L, scan, jnp.int32(0))

        def emit_window(o_k, base, width, cnt):
            def emit(q, cnt_q):
                gl = q * L + lanes
                mvv = mv_v[pl.ds(q * L, L)]
                hit = jnp.logical_and(
                    gl < nmatch,
                    jnp.logical_and(mvv >= o_k, mvv < o_k + width))

                def on_hit():
                    key = jnp.where(hit, 32, 0) + (15 - lanes)
                    mv_s = plsc.sort_key_val(key, mvv, descending=True)[1]
                    mi_s = plsc.sort_key_val(
                        key, mi_v[pl.ds(q * L, L)], descending=True)[1]
                    nh = plsc.all_reduce_population_count(hit)[0]
                    colbase = mv_s - o_k + base
                    for r in range(L):
                        @pl.when(r < nh)
                        def _():
                            col = lane_bcast(colbase, r)
                            slot = lax.rem(cnt_q + r, NBUF)
                            for j in range(D // L):
                                rows_v[slot, pl.ds(L * j, L)] = (
                                    plsc.load_gather(
                                        stage_v, [L * j + lanes, col]))
                            pltpu.async_copy(
                                rows_v.at[slot], out_hbm.at[mi_s[r]], sem_o)
                    return cnt_q + nh
                return lax.cond(jnp.any(hit), on_hit, lambda: cnt_q)

            nq = lax.div(nmatch + (L - 1), jnp.int32(L))
            return lax.fori_loop(0, nq, emit, cnt)

        def drain_to(target, drained):
            def drain(_, c):
                pltpu.make_async_copy(
                    rows_v.at[0], out_hbm.at[0], sem_o).wait()
                return c
            lax.fori_loop(0, jnp.maximum(target - drained, 0), drain,
                          jnp.int32(0))
            return jnp.maximum(target, drained)

        def window(k, carry):
            cnt, drained = carry
            wait_stage()

            @pl.when(k + 2 < n_win)
            def _():
                start(k + 2)
            cnt2 = emit_window(win_off(k), lax.rem(k, 2) * WLANES,
                               WLANES, cnt)
            drained2 = drain_to(cnt2 - NBUF // 2, drained)
            return cnt2, drained2

        cnt_main, drained_main = lax.fori_loop(
            0, n_win, window, (jnp.int32(0), jnp.int32(0)))

        tail = (VOCAB // 128) * 128

        def with_tail():
            tailo = pl.multiple_of(hi * 0 + tail, 128)
            for cb in range(D // 8):
                pltpu.async_copy(
                    table_hbm.at[pl.ds(cb * 8, 8), pl.ds(tailo, 128)],
                    stage_v.at[pl.ds(cb * 8, 8), pl.ds(0, 128)], sem_s)
            for cb in range(D // 8):
                pltpu.make_async_copy(
                    table_hbm.at[pl.ds(0, 8), pl.ds(tailo, 128)],
                    stage_v.at[pl.ds(0, 8), pl.ds(0, 128)], sem_s).wait()
            return emit_window(jnp.int32(tail), 0, 128, cnt_main)

        cnt_final = lax.cond(wid == nw - 1, with_tail, lambda: cnt_main)
        drain_to(cnt_final, drained_main)

    return sc_kernel(idx, posrow, table_t)
